# Initial kernel scaffold; baseline (speedup 1.0000x reference)
import jax
import jax.numpy as jnp
from jax.experimental import pallas as pl


def kernel(x, w_packed, b_packed, min_value, max_value):
    raise NotImplementedError("write your pallas kernel here")


# speedup vs baseline: 1.0581x; 1.0581x over previous
import jax
import jax.numpy as jnp
from jax.experimental import pallas as pl
from jax.experimental.pallas import tpu as pltpu

_IN = 16
_OUT = 32
_BR = 4096


def _fused_body(minmax_ref, x_ref, w_ref, b_ref, o_ref):
    w = w_ref[0:_IN, 0:_OUT]
    acc = jnp.dot(x_ref[...], w, preferred_element_type=jnp.float32)
    acc = acc + b_ref[0:1, 0:_OUT]
    o_ref[...] = jnp.minimum(jnp.maximum(acc, minmax_ref[0]), minmax_ref[1])


@jax.jit
def kernel(x, w_packed, b_packed, min_value, max_value):
    B = x.shape[0]
    minmax = jnp.stack([jnp.asarray(min_value, jnp.float32).reshape(()),
                        jnp.asarray(max_value, jnp.float32).reshape(())])
    nb = pl.cdiv(B, _BR)
    return pl.pallas_call(
        _fused_body,
        out_shape=jax.ShapeDtypeStruct((B, _OUT), jnp.float32),
        grid=(nb,),
        in_specs=[
            pl.BlockSpec(memory_space=pltpu.SMEM),
            pl.BlockSpec((_BR, _IN), lambda i: (i, 0)),
            pl.BlockSpec((128, 256), lambda i: (0, 0)),
            pl.BlockSpec((1, 256), lambda i: (0, 0)),
        ],
        out_specs=pl.BlockSpec((_BR, _OUT), lambda i: (i, 0)),
        compiler_params=pltpu.CompilerParams(
            dimension_semantics=("parallel",)),
    )(minmax, x, w_packed, b_packed)


# speedup vs baseline: 12.9235x; 12.2136x over previous
import jax
import jax.numpy as jnp
from jax.experimental import pallas as pl
from jax.experimental.pallas import tpu as pltpu

_IN = 16
_OUT = 32
_BC = 32768


def _fused_body(minmax_ref, x_ref, p_ref, o_ref):
    wt = p_ref[:, 0:_IN]
    b = p_ref[:, _IN:_IN + 1]
    acc = jnp.dot(wt, x_ref[...], preferred_element_type=jnp.float32)
    acc = acc + b
    o_ref[...] = jnp.minimum(jnp.maximum(acc, minmax_ref[0]), minmax_ref[1])


@jax.jit
def kernel(x, w_packed, b_packed, min_value, max_value):
    B = x.shape[0]
    minmax = jnp.stack([jnp.asarray(min_value, jnp.float32).reshape(()),
                        jnp.asarray(max_value, jnp.float32).reshape(())])
    params = jnp.concatenate(
        [w_packed[:_IN, :_OUT].T, b_packed[0:1, :_OUT].T], axis=1)

    xt = x.T
    nc = pl.cdiv(B, _BC)
    out_t = pl.pallas_call(
        _fused_body,
        out_shape=jax.ShapeDtypeStruct((_OUT, B), jnp.float32),
        grid=(nc,),
        in_specs=[
            pl.BlockSpec(memory_space=pltpu.SMEM),
            pl.BlockSpec((_IN, _BC), lambda i: (0, i)),
            pl.BlockSpec((_OUT, _IN + 1), lambda i: (0, 0)),
        ],
        out_specs=pl.BlockSpec((_OUT, _BC), lambda i: (0, i)),
        compiler_params=pltpu.CompilerParams(
            dimension_semantics=("parallel",)),
    )(minmax, xt, params)
    return out_t.T
